# baseline (speedup 1.0000x reference)
import functools

import jax
import jax.numpy as jnp
from jax import lax
from jax.experimental import pallas as pl
from jax.experimental.pallas import tpu as pltpu
from jax.experimental.pallas import tpu_sc as plsc

_D = 16
_NC = 2
_NS = 16
_NW = _NC * _NS


@jax.jit
def _split_layer(inputs, table):
    batch, cars = inputs.shape
    n = batch * cars
    rows_per_w = batch // _NW
    n_per_w = n // _NW
    mesh = plsc.VectorSubcoreMesh(core_axis_name="c", subcore_axis_name="s")

    @functools.partial(
        pl.kernel,
        mesh=mesh,
        out_type=jax.ShapeDtypeStruct((n,), jnp.int32),
        scratch_types=[
            pltpu.VMEM((rows_per_w, cars), jnp.int32),
            pltpu.VMEM((n_per_w,), jnp.int32),
        ],
    )
    def flatten_kernel(idx_hbm, flat_hbm, idx_v, flat_v):
        wid = lax.axis_index("s") * _NC + lax.axis_index("c")
        pltpu.sync_copy(idx_hbm.at[pl.ds(wid * rows_per_w, rows_per_w)], idx_v)

        def body(r, carry):
            flat_v[pl.ds(r * cars, _D)] = idx_v[r, pl.ds(0, _D)]
            flat_v[pl.ds(r * cars + cars - _D, _D)] = (
                idx_v[r, pl.ds(cars - _D, _D)])
            return carry

        lax.fori_loop(0, rows_per_w, body, 0)
        pltpu.sync_copy(flat_v, flat_hbm.at[pl.ds(wid * n_per_w, n_per_w)])

    @functools.partial(
        pl.kernel,
        mesh=mesh,
        compiler_params=pltpu.CompilerParams(use_tc_tiling_on_sc=False),
        out_type=jax.ShapeDtypeStruct((n, _D), jnp.float32),
        scratch_types=[
            pltpu.VMEM((n_per_w,), jnp.int32),
            pltpu.VMEM((n_per_w, _D), jnp.float32),
            pltpu.SemaphoreType.DMA,
        ],
    )
    def gather_kernel(flat_hbm, table_hbm, out_hbm, idx_v, rows_v, sem):
        wid = lax.axis_index("s") * _NC + lax.axis_index("c")
        base = wid * n_per_w
        pltpu.sync_copy(flat_hbm.at[pl.ds(base, n_per_w)], idx_v)
        pltpu.async_copy(table_hbm.at[idx_v], rows_v, sem).wait()
        pltpu.sync_copy(rows_v, out_hbm.at[pl.ds(base, n_per_w)])

    flat = flatten_kernel(inputs)
    rows = gather_kernel(flat, table)
    return rows.reshape(batch, 1, cars * _D)


def kernel(inputs, table):
    return _split_layer(inputs, table)
